# Initial kernel scaffold; baseline (speedup 1.0000x reference)
import jax
import jax.numpy as jnp
from jax.experimental import pallas as pl


def kernel(data0):
    raise NotImplementedError("write your pallas kernel here")


# speedup vs baseline: 1.8078x; 1.8078x over previous
import jax
import jax.numpy as jnp
from jax import lax
from jax.experimental import pallas as pl

_INPUT_SIZE = 64
_N_FRAMES = 2048
_N_RAW = 543 * 3
_LIPS = [61, 185, 40, 39, 37, 0, 267, 269, 270, 409, 291, 146, 91, 181, 84,
         17, 314, 405, 321, 375, 78, 191, 80, 81, 82, 13, 312, 311, 310, 415,
         95, 88, 178, 87, 14, 317, 402, 318, 324, 308]
_HANDS = list(range(468, 489)) + list(range(522, 543))
_LANDMARKS = _LIPS + _HANDS
_N_COLS = len(_LANDMARKS)
_LFLAT = [3 * l + d for l in _LANDMARKS for d in range(3)]

_HI = jax.lax.Precision.HIGHEST


def _preprocess_kernel(d_ref, lflat_ref, out_ref, nef_ref):
    D = d_ref[...]

    col = lax.broadcasted_iota(jnp.int32, (1, _N_RAW), 1)
    hand_cols = (((col >= 468 * 3) & (col < 489 * 3)) |
                 ((col >= 522 * 3) & (col < 543 * 3))).astype(jnp.float32)
    hs = jnp.sum(D * hand_cols, axis=1, keepdims=True)
    m = (hs > 0).astype(jnp.float32)

    r_i = lax.broadcasted_iota(jnp.int32, (_N_FRAMES, _N_FRAMES), 0)
    c_i = lax.broadcasted_iota(jnp.int32, (_N_FRAMES, _N_FRAMES), 1)
    tri = (r_i >= c_i).astype(jnp.float32)
    p_incl = lax.dot_general(tri, m, (((1,), (0,)), ((), ())),
                             precision=_HI)
    p = p_incl - m
    n = lax.slice(p_incl, (_N_FRAMES - 1, 0), (_N_FRAMES, 1))

    pool = jnp.floor((2.0 * n + (_INPUT_SIZE - 1)) / _INPUT_SIZE)
    q = pool + 1.0
    pad_left = jnp.floor((pool * _INPUT_SIZE - 2.0 * n) / 2.0) + _INPUT_SIZE // 2
    total = _INPUT_SIZE * q

    i_row = lax.broadcasted_iota(
        jnp.int32, (1, _INPUT_SIZE), 1).astype(jnp.float32)
    a = jnp.where(p == 0, 0.0, 2.0 * p + pad_left)
    b = jnp.where(p == n - 1.0, total, 2.0 * p + 2.0 + pad_left)
    lo = jnp.maximum(a, i_row * q)
    hi = jnp.minimum(b, (i_row + 1.0) * q)
    w_long = jnp.maximum(hi - lo, 0.0) / q
    w_short = (p == i_row).astype(jnp.float32)
    wt = m * jnp.where(n < _INPUT_SIZE, w_short, w_long)

    pooled = lax.dot_general(wt, D, (((0,), (0,)), ((), ())), precision=_HI)

    sel_r = lax.broadcasted_iota(jnp.int32, (_N_RAW, len(_LFLAT)), 0)
    sel = (sel_r == lflat_ref[...]).astype(jnp.float32)
    out_ref[...] = lax.dot_general(pooled, sel, (((1,), (0,)), ((), ())),
                                   precision=_HI)

    f_col = lax.broadcasted_iota(
        jnp.int32, (_N_FRAMES, 1), 0).astype(jnp.float32)
    nef = jnp.sum(wt * f_col, axis=0, keepdims=True)
    nef_ref[...] = jnp.where((i_row < n) | (n >= _INPUT_SIZE), nef, -1.0)


def kernel(data0):
    D = data0.reshape(_N_FRAMES, _N_RAW)
    lflat = jnp.asarray(_LFLAT, dtype=jnp.int32).reshape(1, len(_LFLAT))
    out, nef = pl.pallas_call(
        _preprocess_kernel,
        out_shape=(
            jax.ShapeDtypeStruct((_INPUT_SIZE, len(_LFLAT)), jnp.float32),
            jax.ShapeDtypeStruct((1, _INPUT_SIZE), jnp.float32),
        ),
    )(D, lflat)
    return out.reshape(_INPUT_SIZE, _N_COLS, 3), nef.reshape(_INPUT_SIZE)


# speedup vs baseline: 11.3885x; 6.2996x over previous
import jax
import jax.numpy as jnp
from jax import lax
from jax.experimental import pallas as pl

_INPUT_SIZE = 64
_N_FRAMES = 2048
_N_RAW = 543 * 3
_LIPS = [61, 185, 40, 39, 37, 0, 267, 269, 270, 409, 291, 146, 91, 181, 84,
         17, 314, 405, 321, 375, 78, 191, 80, 81, 82, 13, 312, 311, 310, 415,
         95, 88, 178, 87, 14, 317, 402, 318, 324, 308]
_HANDS = list(range(468, 489)) + list(range(522, 543))
_LANDMARKS = _LIPS + _HANDS
_N_COLS = len(_LANDMARKS)
_TGT = [543 * d + l for d in range(3) for l in _LANDMARKS]


def _preprocess_kernel(dt_ref, tgt_ref, out_ref, nef_ref):
    DT = dt_ref[...]

    hs = jnp.zeros((1, _N_FRAMES), jnp.float32)
    for d in range(3):
        base = 543 * d
        hs = hs + jnp.sum(DT[base + 468:base + 489, :], axis=0, keepdims=True)
        hs = hs + jnp.sum(DT[base + 522:base + 543, :], axis=0, keepdims=True)
    m = (hs > 0).astype(jnp.float32)

    p_incl = m
    k = 1
    while k < _N_FRAMES:
        shifted = jnp.concatenate(
            [jnp.zeros((1, k), jnp.float32), p_incl[:, :_N_FRAMES - k]], axis=1)
        p_incl = p_incl + shifted
        k *= 2
    p = p_incl - m
    n = lax.slice(p_incl, (0, _N_FRAMES - 1), (1, _N_FRAMES))

    pool = jnp.floor((2.0 * n + (_INPUT_SIZE - 1)) / _INPUT_SIZE)
    q = pool + 1.0
    pad_left = jnp.floor((pool * _INPUT_SIZE - 2.0 * n) / 2.0) + _INPUT_SIZE // 2
    total = _INPUT_SIZE * q

    i_col = lax.broadcasted_iota(
        jnp.int32, (_INPUT_SIZE, 1), 0).astype(jnp.float32)
    a = jnp.where(p == 0, 0.0, 2.0 * p + pad_left)
    b = jnp.where(p == n - 1.0, total, 2.0 * p + 2.0 + pad_left)
    lo = jnp.maximum(a, i_col * q)
    hi = jnp.minimum(b, (i_col + 1.0) * q)
    w_long = jnp.maximum(hi - lo, 0.0) / q
    w_short = (p == i_col).astype(jnp.float32)
    wt = m * jnp.where(n < _INPUT_SIZE, w_short, w_long)

    wfi = wt.T
    pooled = lax.dot_general(DT, wfi, (((1,), (0,)), ((), ())))

    sel_c = lax.broadcasted_iota(jnp.int32, (len(_TGT), _N_RAW), 1)
    sel = (sel_c == tgt_ref[...]).astype(jnp.float32)
    out_ref[...] = lax.dot_general(sel, pooled, (((1,), (0,)), ((), ())))

    f_row = lax.broadcasted_iota(
        jnp.int32, (1, _N_FRAMES), 1).astype(jnp.float32)
    nef = jnp.sum(wt * f_row, axis=1, keepdims=True)
    nef_ref[...] = jnp.where((i_col < n) | (n >= _INPUT_SIZE), nef, -1.0)


def kernel(data0):
    DT = data0.transpose(2, 1, 0).reshape(_N_RAW, _N_FRAMES)
    tgt = jnp.asarray(_TGT, dtype=jnp.int32).reshape(len(_TGT), 1)
    out, nef = pl.pallas_call(
        _preprocess_kernel,
        out_shape=(
            jax.ShapeDtypeStruct((len(_TGT), _INPUT_SIZE), jnp.float32),
            jax.ShapeDtypeStruct((_INPUT_SIZE, 1), jnp.float32),
        ),
    )(DT, tgt)
    return (out.reshape(3, _N_COLS, _INPUT_SIZE).transpose(2, 1, 0),
            nef.reshape(_INPUT_SIZE))


# speedup vs baseline: 12.4727x; 1.0952x over previous
import jax
import jax.numpy as jnp
from jax import lax
from jax.experimental import pallas as pl
from jax.experimental.pallas import tpu as pltpu

_INPUT_SIZE = 64
_N_FRAMES = 2048
_N_RAW = 543 * 3
_LIPS = [61, 185, 40, 39, 37, 0, 267, 269, 270, 409, 291, 146, 91, 181, 84,
         17, 314, 405, 321, 375, 78, 191, 80, 81, 82, 13, 312, 311, 310, 415,
         95, 88, 178, 87, 14, 317, 402, 318, 324, 308]
_HANDS = list(range(468, 489)) + list(range(522, 543))
_LANDMARKS = _LIPS + _HANDS
_N_COLS = 82
_N_SEL = 3 * _N_COLS
_TAIL0 = 1624
_N_TAIL = _N_RAW - _TAIL0

_RUNS = []
_G_ROW = {}
_GTOT = 0
_BANDS = [(543 * _d + _s, 543 * _d + _s + 21)
          for _d in range(3) for _s in (468, 522)]
_WINS = sorted({(543 * _d + _l) // 8
                for _d in range(3) for _l in _LANDMARKS
                if 543 * _d + _l < _TAIL0})
_i = 0
while _i < len(_WINS):
    _j = _i
    while _j + 1 < len(_WINS) and _WINS[_j + 1] == _WINS[_j] + 1:
        _j += 1
    _start, _nr = _WINS[_i] * 8, (_WINS[_j] - _WINS[_i] + 1) * 8
    _is_hand = any(_start < hi and lo < _start + _nr for lo, hi in _BANDS)
    _RUNS.append((_start, _nr, _GTOT, _is_hand))
    for _r in range(_start, _start + _nr):
        _G_ROW[_r] = _GTOT + _r - _start
    _GTOT += _nr
    _i = _j + 1
_G_TAIL = _GTOT
for _k in range(_N_TAIL):
    _G_ROW[_TAIL0 + _k] = _G_TAIL + _k
_GTOT += 8
_HAND_SEGS = []
for _lo, _hi in _BANDS:
    _r = _lo
    while _r < _hi:
        _g0, _ln = _G_ROW[_r], 1
        while _r + _ln < _hi and _G_ROW[_r + _ln] == _g0 + _ln:
            _ln += 1
        _HAND_SEGS.append((_g0, _ln))
        _r += _ln
_TGT_G = [_G_ROW[543 * _d + _l] for _d in range(3) for _l in _LANDMARKS]


def _preprocess_kernel(dt_ref, tail_ref, tgt_ref, out_ref, nef_ref,
                       g_ref, sem_h, sem_l):
    copies = []
    for start, nr, g0, is_hand in sorted(_RUNS, key=lambda t: not t[3]):
        c = pltpu.make_async_copy(dt_ref.at[start:start + nr, :],
                                  g_ref.at[g0:g0 + nr, :],
                                  sem_h if is_hand else sem_l)
        c.start()
        copies.append((c, is_hand))
    g_ref[_G_TAIL:_G_TAIL + 8, :] = jnp.zeros((8, _N_FRAMES), jnp.float32)
    g_ref[_G_TAIL:_G_TAIL + _N_TAIL, :] = tail_ref[...]
    for c, is_hand in copies:
        if is_hand:
            c.wait()

    hs = jnp.zeros((1, _N_FRAMES), jnp.float32)
    for g0, ln in _HAND_SEGS:
        hs = hs + jnp.sum(g_ref[g0:g0 + ln, :], axis=0, keepdims=True)
    m = (hs > 0).astype(jnp.float32)

    p_incl = m
    k = 1
    while k < _N_FRAMES:
        shifted = jnp.concatenate(
            [jnp.zeros((1, k), jnp.float32), p_incl[:, :_N_FRAMES - k]], axis=1)
        p_incl = p_incl + shifted
        k *= 2
    p = p_incl - m
    n = lax.slice(p_incl, (0, _N_FRAMES - 1), (1, _N_FRAMES))

    pool = jnp.floor((2.0 * n + (_INPUT_SIZE - 1)) / _INPUT_SIZE)
    q = pool + 1.0
    pad_left = jnp.floor((pool * _INPUT_SIZE - 2.0 * n) / 2.0) + _INPUT_SIZE // 2
    total = _INPUT_SIZE * q

    i_col = lax.broadcasted_iota(
        jnp.int32, (_INPUT_SIZE, 1), 0).astype(jnp.float32)
    a = jnp.where(p == 0, 0.0, 2.0 * p + pad_left)
    b = jnp.where(p == n - 1.0, total, 2.0 * p + 2.0 + pad_left)
    lo = jnp.maximum(a, i_col * q)
    hi = jnp.minimum(b, (i_col + 1.0) * q)
    w_long = jnp.maximum(hi - lo, 0.0) / q
    w_short = (p == i_col).astype(jnp.float32)
    wt = m * jnp.where(n < _INPUT_SIZE, w_short, w_long)

    f_row = lax.broadcasted_iota(
        jnp.int32, (1, _N_FRAMES), 1).astype(jnp.float32)
    nef = jnp.sum(wt * f_row, axis=1, keepdims=True)
    nef_ref[...] = jnp.where((i_col < n) | (n >= _INPUT_SIZE), nef, -1.0)

    for c, is_hand in copies:
        if not is_hand:
            c.wait()
    wfi = wt.T
    pooled = lax.dot_general(g_ref[...], wfi, (((1,), (0,)), ((), ())))
    sel_c = lax.broadcasted_iota(jnp.int32, (_N_SEL, _GTOT), 1)
    sel = (sel_c == tgt_ref[...]).astype(jnp.float32)
    out_ref[...] = lax.dot_general(sel, pooled, (((1,), (0,)), ((), ())))


def kernel(data0):
    DT = data0.transpose(2, 1, 0).reshape(_N_RAW, _N_FRAMES)
    tail = lax.slice(DT, (_TAIL0, 0), (_N_RAW, _N_FRAMES))
    tgt = jnp.asarray(_TGT_G, dtype=jnp.int32).reshape(_N_SEL, 1)
    out, nef = pl.pallas_call(
        _preprocess_kernel,
        in_specs=[pl.BlockSpec(memory_space=pl.ANY),
                  pl.BlockSpec(memory_space=pltpu.VMEM),
                  pl.BlockSpec(memory_space=pltpu.VMEM)],
        out_shape=(
            jax.ShapeDtypeStruct((_N_SEL, _INPUT_SIZE), jnp.float32),
            jax.ShapeDtypeStruct((_INPUT_SIZE, 1), jnp.float32),
        ),
        scratch_shapes=[
            pltpu.VMEM((_GTOT, _N_FRAMES), jnp.float32),
            pltpu.SemaphoreType.DMA,
            pltpu.SemaphoreType.DMA,
        ],
    )(DT, tail, tgt)
    return (out.reshape(3, _N_COLS, _INPUT_SIZE).transpose(2, 1, 0),
            nef.reshape(_INPUT_SIZE))
